# Initial kernel scaffold; baseline (speedup 1.0000x reference)
import jax
import jax.numpy as jnp
from jax.experimental import pallas as pl


def kernel(x, edge_index, W1, W2):
    raise NotImplementedError("write your pallas kernel here")


# speedup vs baseline: 9.2577x; 9.2577x over previous
import functools

import jax
import jax.numpy as jnp
from jax import lax
from jax.experimental import pallas as pl
from jax.experimental.pallas import tpu as pltpu
from jax.experimental.pallas import tpu_sc as plsc

N = 10000
E = 160000
D_IN = 256
D_OUT = 64

N_PAD = 10240
SLAB = N_PAD // 16
K = 128
EPT = E // 16
NB = (EPT + K - 1) // K
EPT_PAD = NB * K


def _deg_body(dst_hbm, zeros_hbm, out_hbm, dstbuf, onesbuf, deg_sh):
    c = lax.axis_index("c")
    w = lax.axis_index("s")
    for t in range(SLAB // K):
        pltpu.sync_copy(zeros_hbm, deg_sh.at[pl.ds(w * SLAB + t * K, K)])
    pltpu.sync_copy(dst_hbm.at[w], dstbuf)
    for i in range(K // 16):
        onesbuf[pl.ds(16 * i, 16)] = jnp.full((16,), 1.0, jnp.float32)
    plsc.subcore_barrier()

    def body(j, _):
        pltpu.sync_copy(onesbuf, deg_sh.at[dstbuf.at[j]], add=True)
        return _

    lax.fori_loop(0, NB, body, 0)
    plsc.subcore_barrier()
    pltpu.sync_copy(deg_sh.at[pl.ds(w * SLAB, SLAB)],
                    out_hbm.at[c, pl.ds(w * SLAB, SLAB)])


def _make_deg_kernel():
    mesh = plsc.VectorSubcoreMesh(core_axis_name="c", subcore_axis_name="s")
    return pl.kernel(
        _deg_body,
        out_type=jax.ShapeDtypeStruct((2, N_PAD), jnp.float32),
        mesh=mesh,
        scratch_types=[
            pltpu.VMEM((NB, K), jnp.int32),
            pltpu.VMEM((K,), jnp.float32),
            pltpu.VMEM_SHARED((N_PAD,), jnp.float32),
        ],
    )


def _agg1_body(tab_hbm, src_hbm, dst_hbm, zeros_hbm, out_hbm,
               srcbuf, dstbuf, rows, acc_sh, sem):
    c = lax.axis_index("c")
    w = lax.axis_index("s")
    for t in range(SLAB // K):
        pltpu.sync_copy(zeros_hbm, acc_sh.at[pl.ds(w * SLAB + t * K, K)])
    pltpu.sync_copy(src_hbm.at[c, w], srcbuf)
    pltpu.sync_copy(dst_hbm.at[w], dstbuf)
    plsc.subcore_barrier()

    def body(j, _):
        pltpu.async_copy(tab_hbm.at[srcbuf.at[j]], rows, sem).wait()
        pltpu.sync_copy(rows, acc_sh.at[dstbuf.at[j]], add=True)
        return _

    lax.fori_loop(0, NB, body, 0)
    plsc.subcore_barrier()
    pltpu.sync_copy(acc_sh.at[pl.ds(w * SLAB, SLAB)],
                    out_hbm.at[c, pl.ds(w * SLAB, SLAB)])


def _make_agg1_kernel():
    mesh = plsc.VectorSubcoreMesh(core_axis_name="c", subcore_axis_name="s")
    return pl.kernel(
        _agg1_body,
        out_type=jax.ShapeDtypeStruct((2, N_PAD, 128), jnp.float32),
        mesh=mesh,
        scratch_types=[
            pltpu.VMEM((NB, K), jnp.int32),
            pltpu.VMEM((NB, K), jnp.int32),
            pltpu.VMEM((K, 128), jnp.float32),
            pltpu.VMEM_SHARED((N_PAD, 128), jnp.float32),
            pltpu.SemaphoreType.DMA,
        ],
    )


NB2 = (E // 32 + K - 1) // K


def _agg2_body(tab_hbm, src_hbm, dst_hbm, zeros_hbm, out_hbm,
               srcbuf, dstbuf, rows, acc_sh, sem):
    c = lax.axis_index("c")
    w = lax.axis_index("s")
    chunk = c * 16 + w
    for t in range(SLAB // K):
        pltpu.sync_copy(zeros_hbm, acc_sh.at[pl.ds(w * SLAB + t * K, K)])
    pltpu.sync_copy(src_hbm.at[chunk], srcbuf)
    pltpu.sync_copy(dst_hbm.at[chunk], dstbuf)
    plsc.subcore_barrier()

    def body(j, _):
        pltpu.async_copy(tab_hbm.at[srcbuf.at[j]], rows, sem).wait()
        pltpu.sync_copy(rows, acc_sh.at[dstbuf.at[j]], add=True)
        return _

    lax.fori_loop(0, NB2, body, 0)
    plsc.subcore_barrier()
    pltpu.sync_copy(acc_sh.at[pl.ds(w * SLAB, SLAB)],
                    out_hbm.at[c, pl.ds(w * SLAB, SLAB)])


def _make_agg2_kernel():
    mesh = plsc.VectorSubcoreMesh(core_axis_name="c", subcore_axis_name="s")
    return pl.kernel(
        _agg2_body,
        out_type=jax.ShapeDtypeStruct((2, N_PAD, 128), jnp.float32),
        mesh=mesh,
        scratch_types=[
            pltpu.VMEM((NB2, K), jnp.int32),
            pltpu.VMEM((NB2, K), jnp.int32),
            pltpu.VMEM((K, 128), jnp.float32),
            pltpu.VMEM_SHARED((N_PAD, 128), jnp.float32),
            pltpu.SemaphoreType.DMA,
        ],
    )


BR = 2560


def _tc1_body(deg_ref, x_ref, w1_ref, out_ref):
    dinv = lax.rsqrt(deg_ref[...] + 1.0)
    h = jnp.dot(x_ref[...], w1_ref[...], preferred_element_type=jnp.float32)
    g = h * dinv
    out_ref[0] = g[:, :128]
    out_ref[1] = g[:, 128:]


def _tc2_body(deg_ref, s_ref, g_ref, w2_ref, out_ref):
    dinv = lax.rsqrt(deg_ref[...] + 1.0)
    oa = jnp.maximum((s_ref[0] + g_ref[0]) * dinv, 0.0)
    ob = jnp.maximum((s_ref[1] + g_ref[1]) * dinv, 0.0)
    h2 = (jnp.dot(oa, w2_ref[:128], preferred_element_type=jnp.float32)
          + jnp.dot(ob, w2_ref[128:], preferred_element_type=jnp.float32))
    g2 = h2 * dinv
    out_ref[...] = jnp.pad(g2, ((0, 0), (0, 64)))


def _tc3_body(deg_ref, s_ref, g_ref, out_ref):
    dinv = lax.rsqrt(deg_ref[...] + 1.0)
    p = ((s_ref[0] + s_ref[1] + g_ref[...]) * dinv)[:, :D_OUT]
    m = jnp.max(p, axis=1, keepdims=True)
    e = jnp.exp(p - m)
    out_ref[...] = e / jnp.sum(e, axis=1, keepdims=True)


def _tc1(deg, x_pad, W1):
    return pl.pallas_call(
        _tc1_body,
        grid=(N_PAD // BR,),
        in_specs=[
            pl.BlockSpec((BR, 1), lambda b: (b, 0)),
            pl.BlockSpec((BR, D_IN), lambda b: (b, 0)),
            pl.BlockSpec((D_IN, D_IN), lambda b: (0, 0)),
        ],
        out_specs=pl.BlockSpec((2, BR, 128), lambda b: (0, b, 0)),
        out_shape=jax.ShapeDtypeStruct((2, N_PAD, 128), jnp.float32),
    )(deg, x_pad, W1)


def _tc2(deg, s1, g1, W2):
    return pl.pallas_call(
        _tc2_body,
        grid=(N_PAD // BR,),
        in_specs=[
            pl.BlockSpec((BR, 1), lambda b: (b, 0)),
            pl.BlockSpec((2, BR, 128), lambda b: (0, b, 0)),
            pl.BlockSpec((2, BR, 128), lambda b: (0, b, 0)),
            pl.BlockSpec((D_IN, D_OUT), lambda b: (0, 0)),
        ],
        out_specs=pl.BlockSpec((BR, 128), lambda b: (b, 0)),
        out_shape=jax.ShapeDtypeStruct((N_PAD, 128), jnp.float32),
    )(deg, s1, g1, W2)


def _tc3(deg, s2, g2):
    return pl.pallas_call(
        _tc3_body,
        grid=(N_PAD // BR,),
        in_specs=[
            pl.BlockSpec((BR, 1), lambda b: (b, 0)),
            pl.BlockSpec((2, BR, 128), lambda b: (0, b, 0)),
            pl.BlockSpec((BR, 128), lambda b: (b, 0)),
        ],
        out_specs=pl.BlockSpec((BR, D_OUT), lambda b: (b, 0)),
        out_shape=jax.ShapeDtypeStruct((N_PAD, D_OUT), jnp.float32),
    )(deg, s2, g2)


@jax.jit
def kernel(x, edge_index, W1, W2):
    src = edge_index[0]
    dst = edge_index[1]

    x_pad = jnp.zeros((N_PAD, D_IN), jnp.float32).at[:N].set(x)

    pad_e = jnp.full((16, EPT_PAD - EPT), N, jnp.int32)
    src_t = jnp.concatenate([src.reshape(16, EPT), pad_e], axis=1)
    dst_t = jnp.concatenate([dst.reshape(16, EPT), pad_e], axis=1)
    src_t = src_t.reshape(16, NB, K)
    dst_t = dst_t.reshape(16, NB, K)
    src2 = jnp.stack([src_t, src_t + N_PAD])

    ept2 = E // 32
    pad_e2 = jnp.full((32, NB2 * K - ept2), N, jnp.int32)
    src_e = jnp.concatenate([src.reshape(32, ept2), pad_e2], axis=1).reshape(32, NB2, K)
    dst_e = jnp.concatenate([dst.reshape(32, ept2), pad_e2], axis=1).reshape(32, NB2, K)

    zeros_vec = jnp.zeros((K,), jnp.float32)
    zeros128 = jnp.zeros((K, 128), jnp.float32)

    deg2 = _make_deg_kernel()(dst_t, zeros_vec)
    deg = deg2[0].reshape(N_PAD, 1)

    g1 = _tc1(deg, x_pad, W1)
    s1 = _make_agg1_kernel()(g1.reshape(2 * N_PAD, 128), src2, dst_t, zeros128)
    g2 = _tc2(deg, s1, g1, W2)
    s2 = _make_agg2_kernel()(g2, src_e, dst_e, zeros128)
    out = _tc3(deg, s2, g2)
    return out[:N]


# speedup vs baseline: 9.7171x; 1.0496x over previous
import functools

import jax
import jax.numpy as jnp
from jax import lax
from jax.experimental import pallas as pl
from jax.experimental.pallas import tpu as pltpu
from jax.experimental.pallas import tpu_sc as plsc

N = 10000
E = 160000
D_IN = 256
D_OUT = 64

N_PAD = 10240
SLAB = N_PAD // 16
K = 128
H = 40

DEG_PAD = N_PAD
DEG_SLAB = DEG_PAD // 16


def _deg_body(dst_hbm, zeros_hbm, out_hbm, dstbuf, onesbuf, deg_sh):
    c = lax.axis_index("c")
    w = lax.axis_index("s")
    for t in range(DEG_SLAB // K):
        pltpu.sync_copy(zeros_hbm, deg_sh.at[pl.ds(w * DEG_SLAB + t * K, K)])
    pltpu.sync_copy(dst_hbm.at[w], dstbuf)
    for i in range(K // 16):
        onesbuf[pl.ds(16 * i, 16)] = jnp.full((16,), 1.0, jnp.float32)
    plsc.subcore_barrier()

    def body(j, _):
        pltpu.sync_copy(onesbuf, deg_sh.at[dstbuf.at[j]], add=True)
        return _

    lax.fori_loop(0, 2 * H, body, 0)
    plsc.subcore_barrier()
    pltpu.sync_copy(deg_sh.at[pl.ds(w * DEG_SLAB, DEG_SLAB)],
                    out_hbm.at[c, pl.ds(w * DEG_SLAB, DEG_SLAB)])


def _make_deg_kernel():
    mesh = plsc.VectorSubcoreMesh(core_axis_name="c", subcore_axis_name="s")
    return pl.kernel(
        _deg_body,
        out_type=jax.ShapeDtypeStruct((2, DEG_PAD), jnp.float32),
        mesh=mesh,
        scratch_types=[
            pltpu.VMEM((2 * H, K), jnp.int32),
            pltpu.VMEM((K,), jnp.float32),
            pltpu.VMEM_SHARED((DEG_PAD,), jnp.float32),
        ],
    )


def _agg_body(phases, tab_hbm, src_hbm, dst_hbm, zeros_hbm, out_hbm,
              srcbuf, dstbuf, r0, r1, acc_sh, gs0, gs1):
    c = lax.axis_index("c")
    w = lax.axis_index("s")
    wid = c * 16 + w
    pltpu.sync_copy(zeros_hbm, acc_sh.at[pl.ds(w * SLAB, SLAB)])
    plsc.subcore_barrier()

    def gather(j, rows, sem):
        pltpu.make_async_copy(tab_hbm.at[srcbuf.at[j]], rows, sem).start()

    def gwait(rows, sem):
        pltpu.make_async_copy(tab_hbm.at[srcbuf.at[0]], rows, sem).wait()

    for p in range(phases):
        pltpu.sync_copy(src_hbm.at[wid, pl.ds(p * H, H)], srcbuf)
        pltpu.sync_copy(dst_hbm.at[wid, pl.ds(p * H, H)], dstbuf)
        gather(0, r0, gs0)

        def body(i, _):
            j0 = 2 * i
            j1 = 2 * i + 1
            gather(j1, r1, gs1)
            gwait(r0, gs0)
            pltpu.sync_copy(r0, acc_sh.at[dstbuf.at[j0]], add=True)
            gather(jnp.where(j1 + 1 < H, j1 + 1, 0), r0, gs0)
            gwait(r1, gs1)
            pltpu.sync_copy(r1, acc_sh.at[dstbuf.at[j1]], add=True)
            return _

        lax.fori_loop(0, H // 2, body, 0)
        gwait(r0, gs0)

    plsc.subcore_barrier()
    pltpu.sync_copy(acc_sh.at[pl.ds(w * SLAB, SLAB)],
                    out_hbm.at[c, pl.ds(w * SLAB, SLAB)])


def _make_agg_kernel(phases):
    mesh = plsc.VectorSubcoreMesh(core_axis_name="c", subcore_axis_name="s")
    return pl.kernel(
        functools.partial(_agg_body, phases),
        out_type=jax.ShapeDtypeStruct((2, N_PAD, 128), jnp.float32),
        mesh=mesh,
        scratch_types=[
            pltpu.VMEM((H, K), jnp.int32),
            pltpu.VMEM((H, K), jnp.int32),
            pltpu.VMEM((K, 128), jnp.float32),
            pltpu.VMEM((K, 128), jnp.float32),
            pltpu.VMEM_SHARED((N_PAD, 128), jnp.float32),
            pltpu.SemaphoreType.DMA,
            pltpu.SemaphoreType.DMA,
        ],
    )


BR = 2560


def _tc1_body(deg_ref, x_ref, w1_ref, out_ref):
    dinv = lax.rsqrt(deg_ref[...] + 1.0)
    h = jnp.dot(x_ref[...], w1_ref[...], preferred_element_type=jnp.float32)
    g = h * dinv
    out_ref[0] = g[:, :128]
    out_ref[1] = g[:, 128:]


def _tc2_body(deg_ref, s_ref, g_ref, w2_ref, out_ref):
    dinv = lax.rsqrt(deg_ref[...] + 1.0)
    oa = jnp.maximum((s_ref[0] + g_ref[0]) * dinv, 0.0)
    ob = jnp.maximum((s_ref[1] + g_ref[1]) * dinv, 0.0)
    h2 = (jnp.dot(oa, w2_ref[:128], preferred_element_type=jnp.float32)
          + jnp.dot(ob, w2_ref[128:], preferred_element_type=jnp.float32))
    g2 = h2 * dinv
    out_ref[...] = jnp.pad(g2, ((0, 0), (0, 64)))


def _tc3_body(deg_ref, s_ref, g_ref, out_ref):
    dinv = lax.rsqrt(deg_ref[...] + 1.0)
    p = ((s_ref[0] + s_ref[1] + g_ref[...]) * dinv)[:, :D_OUT]
    m = jnp.max(p, axis=1, keepdims=True)
    e = jnp.exp(p - m)
    out_ref[...] = e / jnp.sum(e, axis=1, keepdims=True)


def _tc1(deg, x_pad, W1):
    return pl.pallas_call(
        _tc1_body,
        grid=(N_PAD // BR,),
        in_specs=[
            pl.BlockSpec((BR, 1), lambda b: (b, 0)),
            pl.BlockSpec((BR, D_IN), lambda b: (b, 0)),
            pl.BlockSpec((D_IN, D_IN), lambda b: (0, 0)),
        ],
        out_specs=pl.BlockSpec((2, BR, 128), lambda b: (0, b, 0)),
        out_shape=jax.ShapeDtypeStruct((2, N_PAD, 128), jnp.float32),
    )(deg, x_pad, W1)


def _tc2(deg, s1, g1, W2):
    return pl.pallas_call(
        _tc2_body,
        grid=(N_PAD // BR,),
        in_specs=[
            pl.BlockSpec((BR, 1), lambda b: (b, 0)),
            pl.BlockSpec((2, BR, 128), lambda b: (0, b, 0)),
            pl.BlockSpec((2, BR, 128), lambda b: (0, b, 0)),
            pl.BlockSpec((D_IN, D_OUT), lambda b: (0, 0)),
        ],
        out_specs=pl.BlockSpec((BR, 128), lambda b: (b, 0)),
        out_shape=jax.ShapeDtypeStruct((N_PAD, 128), jnp.float32),
    )(deg, s1, g1, W2)


def _tc3(deg, s2, g2):
    return pl.pallas_call(
        _tc3_body,
        grid=(N_PAD // BR,),
        in_specs=[
            pl.BlockSpec((BR, 1), lambda b: (b, 0)),
            pl.BlockSpec((2, BR, 128), lambda b: (0, b, 0)),
            pl.BlockSpec((BR, 128), lambda b: (b, 0)),
        ],
        out_specs=pl.BlockSpec((BR, D_OUT), lambda b: (b, 0)),
        out_shape=jax.ShapeDtypeStruct((N_PAD, D_OUT), jnp.float32),
    )(deg, s2, g2)


@jax.jit
def kernel(x, edge_index, W1, W2):
    src = edge_index[0]
    dst = edge_index[1]

    x_pad = jnp.zeros((N_PAD, D_IN), jnp.float32).at[:N].set(x)

    ept1 = E // 16
    pad1 = jnp.full((16, 2 * H * K - ept1), N, jnp.int32)
    src16 = jnp.concatenate([src.reshape(16, ept1), pad1], axis=1)
    dst16 = jnp.concatenate([dst.reshape(16, ept1), pad1], axis=1)
    src_w1 = jnp.stack([src16, src16 + N_PAD]).reshape(32, 2 * H, K)
    dst_w1 = jnp.stack([dst16, dst16]).reshape(32, 2 * H, K)

    ept2 = E // 32
    pad2 = jnp.full((32, H * K - ept2), N, jnp.int32)
    src_w2 = jnp.concatenate([src.reshape(32, ept2), pad2], axis=1).reshape(32, H, K)
    dst_w2 = jnp.concatenate([dst.reshape(32, ept2), pad2], axis=1).reshape(32, H, K)

    zeros_vec = jnp.zeros((K,), jnp.float32)
    zeros_slab = jnp.zeros((SLAB, 128), jnp.float32)

    deg2 = _make_deg_kernel()(dst_w1[:16], zeros_vec)
    deg = deg2[0, :N_PAD].reshape(N_PAD, 1)

    g1 = _tc1(deg, x_pad, W1)
    s1 = _make_agg_kernel(2)(g1.reshape(2 * N_PAD, 128), src_w1, dst_w1,
                             zeros_slab)
    g2 = _tc2(deg, s1, g1, W2)
    s2 = _make_agg_kernel(1)(g2, src_w2, dst_w2, zeros_slab)
    out = _tc3(deg, s2, g2)
    return out[:N]
